# baseline (speedup 1.0000x reference)
import functools

import jax
import jax.numpy as jnp
from jax import lax
from jax.experimental import pallas as pl
from jax.experimental.pallas import tpu as pltpu
from jax.experimental.pallas import tpu_sc as plsc

VOCAB = 1000000
EMBED_DIM = 32
BATCH = 4096
HIST = 200

_INFO = plsc.get_sparse_core_info()
NC = _INFO.num_cores
NS = _INFO.num_subcores
NW = NC * NS

B_TOTAL = BATCH * HIST
B_PER_W = B_TOTAL // NW
ROWS_PER_GATHER = 128
GATHERS_PER_W = B_PER_W // ROWS_PER_GATHER
CHUNK_GATHERS = 10
CHUNK_ROWS = CHUNK_GATHERS * ROWS_PER_GATHER
N_CHUNKS = GATHERS_PER_W // CHUNK_GATHERS


def _make_gather():
    mesh = plsc.VectorSubcoreMesh(core_axis_name="c", subcore_axis_name="s")

    @functools.partial(
        pl.kernel,
        out_type=jax.ShapeDtypeStruct((B_TOTAL, EMBED_DIM), jnp.float32),
        mesh=mesh,
        scratch_types=[
            pltpu.VMEM((GATHERS_PER_W, ROWS_PER_GATHER), jnp.int32),
            pltpu.VMEM((CHUNK_ROWS, EMBED_DIM), jnp.float32),
            pltpu.SemaphoreType.DMA,
        ],
        compiler_params=pltpu.CompilerParams(use_tc_tiling_on_sc=False),
    )
    def k(table_hbm, idx_hbm, out_hbm, idx_v, rows_v, sem):
        wid = lax.axis_index("s") * NC + lax.axis_index("c")
        pltpu.sync_copy(idx_hbm.at[pl.ds(wid * GATHERS_PER_W, GATHERS_PER_W)],
                        idx_v)
        row_base = wid * B_PER_W

        def chunk_body(c, _):
            descs = []
            for g in range(CHUNK_GATHERS):
                descs.append(pltpu.async_copy(
                    table_hbm.at[idx_v.at[c * CHUNK_GATHERS + g]],
                    rows_v.at[pl.ds(g * ROWS_PER_GATHER, ROWS_PER_GATHER)],
                    sem))
            for d in descs:
                d.wait()
            pltpu.sync_copy(
                rows_v,
                out_hbm.at[pl.ds(row_base + c * CHUNK_ROWS, CHUNK_ROWS)])
            return ()

        lax.fori_loop(0, N_CHUNKS, chunk_body, ())

    return k


_gather = _make_gather()


def kernel(inp, lengths, table):
    del lengths
    idx = jnp.clip(inp.astype(jnp.int32), 0, VOCAB - 1)
    idx = idx.T.reshape(B_TOTAL // ROWS_PER_GATHER, ROWS_PER_GATHER)
    out = _gather(table, idx)
    return out.reshape(HIST, BATCH, EMBED_DIM).transpose(1, 0, 2)
